# baseline (speedup 1.0000x reference)
import jax
import jax.numpy as jnp
from jax import lax
from jax.experimental import pallas as pl
from jax.experimental.pallas import tpu as pltpu
import jax.experimental.pallas.tpu_sc as plsc

_NUM_CORES = 2
_NUM_SUBCORES = 16
_NW = _NUM_CORES * _NUM_SUBCORES
_L = 16

_D = 128
_C = 80
_GRP = _C // _L
_NBUF = 3


def _body(feat_hbm, src_hbm, dst_hbm, out_hbm,
          sidx, didx, bufs, pad, obuf, *sems):
    n_chunks = sidx.shape[0]
    cid = lax.axis_index("c")
    sid = lax.axis_index("s")
    wid = sid * _NUM_CORES + cid

    pltpu.sync_copy(src_hbm.at[wid], sidx)
    pltpu.sync_copy(dst_hbm.at[wid], didx)

    iota = lax.iota(jnp.int32, _L)

    def compute_chunk(g, ubuf, vbuf):
        def grp_body(k, _):
            for e in range(_L):
                row = k * _L + e
                acc = ubuf[row, pl.ds(0, _L)] * vbuf[row, pl.ds(0, _L)]
                for d in range(1, _D // _L):
                    acc = acc + (ubuf[row, pl.ds(d * _L, _L)]
                                 * vbuf[row, pl.ds(d * _L, _L)])
                pad[pl.ds(e * (_L + 1), _L)] = acc
            tot = plsc.load_gather(pad, [iota * (_L + 1)])
            for l in range(1, _L):
                tot = tot + plsc.load_gather(pad, [iota * (_L + 1) + l])
            obuf[g, pl.ds(k * _L, _L)] = tot
            return ()

        lax.fori_loop(0, _GRP, grp_body, (), unroll=False)

    _H = _C // 2

    def start_chunk(g, b):
        pltpu.make_async_copy(
            feat_hbm.at[sidx.at[g, pl.ds(0, _H)]],
            bufs.at[2 * b, pl.ds(0, _H)], sems[2 * b]).start()
        pltpu.make_async_copy(
            feat_hbm.at[sidx.at[g, pl.ds(_H, _H)]],
            bufs.at[2 * b, pl.ds(_H, _H)], sems[2 * b]).start()
        pltpu.make_async_copy(
            feat_hbm.at[didx.at[g, pl.ds(0, _H)]],
            bufs.at[2 * b + 1, pl.ds(0, _H)], sems[2 * b + 1]).start()
        pltpu.make_async_copy(
            feat_hbm.at[didx.at[g, pl.ds(_H, _H)]],
            bufs.at[2 * b + 1, pl.ds(_H, _H)], sems[2 * b + 1]).start()

    for b in range(_NBUF):
        start_chunk(b, b)

    n_iters = (n_chunks + _NBUF - 1) // _NBUF

    def ring(i, _):
        for b in range(_NBUF):
            g = i * _NBUF + b

            @pl.when(g < n_chunks)
            def _process():
                ubuf = bufs.at[2 * b]
                vbuf = bufs.at[2 * b + 1]
                pltpu.make_async_copy(
                    feat_hbm.at[sidx.at[g]], ubuf, sems[2 * b]).wait()
                pltpu.make_async_copy(
                    feat_hbm.at[didx.at[g]], vbuf, sems[2 * b + 1]).wait()
                compute_chunk(g, ubuf, vbuf)
                gn = g + _NBUF

                @pl.when(gn < n_chunks)
                def _refill():
                    start_chunk(gn, b)
        return ()

    lax.fori_loop(0, n_iters, ring, (), unroll=False)
    pltpu.sync_copy(obuf, out_hbm.at[wid])


def kernel(feat, edge_index):
    n_edges = edge_index.shape[1]
    per_w = n_edges // _NW
    n_chunks = per_w // _C
    assert per_w * _NW == n_edges and n_chunks * _C == per_w

    src = edge_index[0].astype(jnp.int32).reshape(_NW, n_chunks, _C)
    dst = edge_index[1].astype(jnp.int32).reshape(_NW, n_chunks, _C)

    mesh = plsc.VectorSubcoreMesh(
        core_axis_name="c", subcore_axis_name="s",
        num_cores=_NUM_CORES, num_subcores=_NUM_SUBCORES)

    run = pl.kernel(
        _body,
        out_type=jax.ShapeDtypeStruct((_NW, n_chunks, _C), jnp.float32),
        mesh=mesh,
        scratch_types=[
            pltpu.VMEM((n_chunks, _C), jnp.int32),
            pltpu.VMEM((n_chunks, _C), jnp.int32),
            pltpu.VMEM((2 * _NBUF, _C, _D), jnp.float32),
            pltpu.VMEM((_L * (_L + 1),), jnp.float32),
            pltpu.VMEM((n_chunks, _C), jnp.float32),
        ] + [pltpu.SemaphoreType.DMA] * (2 * _NBUF),
        compiler_params=pltpu.CompilerParams(needs_layout_passes=False),
    )
    score = run(feat, src, dst)
    return score.reshape(n_edges, 1)


# speedup vs baseline: 1.0198x; 1.0198x over previous
import jax
import jax.numpy as jnp
from jax import lax
from jax.experimental import pallas as pl
from jax.experimental.pallas import tpu as pltpu
import jax.experimental.pallas.tpu_sc as plsc

_NUM_CORES = 2
_NUM_SUBCORES = 16
_NW = _NUM_CORES * _NUM_SUBCORES
_L = 16

_D = 128
_C = 80
_GRP = _C // _L
_NBUF = 3


def _body(feat_hbm, src_hbm, dst_hbm, out_hbm,
          sidx, didx, bufs, pad, obuf, *sems):
    n_chunks = sidx.shape[0]
    cid = lax.axis_index("c")
    sid = lax.axis_index("s")
    wid = sid * _NUM_CORES + cid

    pltpu.sync_copy(src_hbm.at[wid], sidx)
    pltpu.sync_copy(dst_hbm.at[wid], didx)

    iota = lax.iota(jnp.int32, _L)

    def compute_chunk(g, ubuf, vbuf):
        def grp_body(k, _):
            for e in range(_L):
                row = k * _L + e
                acc = ubuf[row, pl.ds(0, _L)] * vbuf[row, pl.ds(0, _L)]
                for d in range(1, _D // _L):
                    acc = acc + (ubuf[row, pl.ds(d * _L, _L)]
                                 * vbuf[row, pl.ds(d * _L, _L)])
                pad[pl.ds(e * (_L + 1), _L)] = acc
            tots = [plsc.load_gather(pad, [iota * (_L + 1) + c])
                    for c in range(4)]
            for l in range(4, _L):
                tots[l % 4] = tots[l % 4] + plsc.load_gather(
                    pad, [iota * (_L + 1) + l])
            obuf[g, pl.ds(k * _L, _L)] = (tots[0] + tots[1]) + (tots[2] + tots[3])
            return ()

        lax.fori_loop(0, _GRP, grp_body, (), unroll=False)

    def start_chunk(g, b):
        pltpu.make_async_copy(
            feat_hbm.at[sidx.at[g]], bufs.at[2 * b], sems[2 * b]).start()
        pltpu.make_async_copy(
            feat_hbm.at[didx.at[g]], bufs.at[2 * b + 1], sems[2 * b + 1]).start()

    for b in range(_NBUF):
        start_chunk(b, b)

    n_iters = (n_chunks + _NBUF - 1) // _NBUF

    def ring(i, _):
        for b in range(_NBUF):
            g = i * _NBUF + b

            @pl.when(g < n_chunks)
            def _process():
                ubuf = bufs.at[2 * b]
                vbuf = bufs.at[2 * b + 1]
                pltpu.make_async_copy(
                    feat_hbm.at[sidx.at[g]], ubuf, sems[2 * b]).wait()
                pltpu.make_async_copy(
                    feat_hbm.at[didx.at[g]], vbuf, sems[2 * b + 1]).wait()
                compute_chunk(g, ubuf, vbuf)
                gn = g + _NBUF

                @pl.when(gn < n_chunks)
                def _refill():
                    start_chunk(gn, b)
        return ()

    lax.fori_loop(0, n_iters, ring, (), unroll=False)
    pltpu.sync_copy(obuf, out_hbm.at[wid])


def kernel(feat, edge_index):
    n_edges = edge_index.shape[1]
    per_w = n_edges // _NW
    n_chunks = per_w // _C
    assert per_w * _NW == n_edges and n_chunks * _C == per_w

    src = edge_index[0].astype(jnp.int32).reshape(_NW, n_chunks, _C)
    dst = edge_index[1].astype(jnp.int32).reshape(_NW, n_chunks, _C)

    mesh = plsc.VectorSubcoreMesh(
        core_axis_name="c", subcore_axis_name="s",
        num_cores=_NUM_CORES, num_subcores=_NUM_SUBCORES)

    run = pl.kernel(
        _body,
        out_type=jax.ShapeDtypeStruct((_NW, n_chunks, _C), jnp.float32),
        mesh=mesh,
        scratch_types=[
            pltpu.VMEM((n_chunks, _C), jnp.int32),
            pltpu.VMEM((n_chunks, _C), jnp.int32),
            pltpu.VMEM((2 * _NBUF, _C, _D), jnp.float32),
            pltpu.VMEM((_L * (_L + 1),), jnp.float32),
            pltpu.VMEM((n_chunks, _C), jnp.float32),
        ] + [pltpu.SemaphoreType.DMA] * (2 * _NBUF),
        compiler_params=pltpu.CompilerParams(needs_layout_passes=False),
    )
    score = run(feat, src, dst)
    return score.reshape(n_edges, 1)
